# Initial kernel scaffold; baseline (speedup 1.0000x reference)
import jax
import jax.numpy as jnp
from jax.experimental import pallas as pl


def kernel(x, edge_index, edge_attr, batch, W1, b1, W2, b2, W3, b3, We1, be1, We2, be2):
    raise NotImplementedError("write your pallas kernel here")


# speedup vs baseline: 8.4808x; 8.4808x over previous
import functools

import jax
import jax.numpy as jnp
from jax import lax
from jax.experimental import pallas as pl
from jax.experimental.pallas import tpu as pltpu
from jax.experimental.pallas import tpu_sc as plsc

N = 10000
E = 320000
D = 128
H = 128
A = 32
G = 16

NC = 2
NS = 16
L = 16

NPT = 640
NPAD = NS * NPT
ET1 = E // NS
KD = 2000
K = 80
NH = NPAD // NC
AR = 5248
ZR = AR // NS
DR = NH // NS

NB = 10
BN = NPAD // NB

_SC_PARAMS = pltpu.CompilerParams(needs_layout_passes=False)


def _ew_body(rowf_ref, colf_ref, attr_ref, p_ref, out_ref):
    p = p_ref[0]
    h = jnp.maximum(rowf_ref[...] * p[0] + colf_ref[...] * p[1]
                    + attr_ref[...] * p[2] + p[3], 0.0)
    out_ref[...] = jax.nn.sigmoid(h * p[4] + p[5])


def _edge_weights(rowf, colf, attrf, params):
    return pl.pallas_call(
        _ew_body,
        out_shape=jax.ShapeDtypeStruct((E // 128, 128), jnp.float32),
    )(rowf, colf, attrf, params)


def _deg_body(ew_hbm, col_hbm, disbc_hbm,
              colv, ewv, acc, tmpv, dredv, drv, dbuf, redsh, dissh, sem):
    cid = lax.axis_index("c")
    sid = lax.axis_index("s")
    zero16 = jnp.zeros((L,), jnp.float32)

    def zg(g, _):
        acc[pl.ds(g * L, L)] = zero16
        return 0
    lax.fori_loop(0, NPAD // L, zg, 0)

    def step(t, _):
        base = sid * ET1 + t * KD
        pltpu.sync_copy(ew_hbm.at[pl.ds(base, KD)], ewv)
        pltpu.sync_copy(col_hbm.at[pl.ds(base, KD)], colv)

        def grp(g, _):
            c16 = colv[pl.ds(g * L, L)]
            w16 = ewv[pl.ds(g * L, L)]
            plsc.addupdate_scatter(acc, [c16], w16)
            return 0
        lax.fori_loop(0, KD // L, grp, 0)
        return 0
    lax.fori_loop(0, ET1 // KD, step, 0)

    pltpu.sync_copy(acc, redsh.at[sid])
    plsc.subcore_barrier()

    def zg2(g, _):
        dredv[pl.ds(g * L, L)] = zero16
        return 0
    lax.fori_loop(0, NPT // L, zg2, 0)
    for t in range(NS):
        pltpu.sync_copy(redsh.at[t, pl.ds(sid * NPT, NPT)], tmpv)

        def addg(g, _):
            dredv[pl.ds(g * L, L)] = (dredv[pl.ds(g * L, L)]
                                      + tmpv[pl.ds(g * L, L)])
            return 0
        lax.fori_loop(0, NPT // L, addg, 0)

    def newt(g, _):
        d = 1.0 + dredv[pl.ds(g * L, L)]
        bits = plsc.bitcast(d, jnp.int32)
        yv = plsc.bitcast(jnp.int32(0x5F3759DF)
                          - lax.shift_right_logical(bits, 1), jnp.float32)
        for _ in range(3):
            yv = yv * (1.5 - 0.5 * d * yv * yv)
        dredv[pl.ds(g * L, L)] = yv
        return 0
    lax.fori_loop(0, NPT // L, newt, 0)

    pltpu.sync_copy(dredv, dissh.at[pl.ds(sid * NPT, NPT)])
    plsc.subcore_barrier()

    obase = cid * NH + sid * DR
    pltpu.sync_copy(dissh.at[pl.ds(obase, DR)], drv)

    def brow(g, _):
        d16 = drv[pl.ds(g * L, L)]
        for j in range(L):
            r = g * L + j
            dj = jnp.full((L,), d16[j], jnp.float32)
            for f in range(D // L):
                dbuf[r, pl.ds(f * L, L)] = dj
        return 0
    lax.fori_loop(0, DR // L, brow, 0)
    pltpu.sync_copy(dbuf, disbc_hbm.at[pl.ds(obase, DR)])


def _dis_broadcast(ew, col):
    mesh = plsc.VectorSubcoreMesh(core_axis_name="c", subcore_axis_name="s",
                                  num_cores=NC, num_subcores=NS)
    return pl.kernel(
        _deg_body,
        out_type=jax.ShapeDtypeStruct((NPAD, D), jnp.float32),
        mesh=mesh,
        scratch_types=[
            pltpu.VMEM((KD,), jnp.int32),
            pltpu.VMEM((KD,), jnp.float32),
            pltpu.VMEM((NPAD,), jnp.float32),
            pltpu.VMEM((NPT,), jnp.float32),
            pltpu.VMEM((NPT,), jnp.float32),
            pltpu.VMEM((DR,), jnp.float32),
            pltpu.VMEM((DR, D), jnp.float32),
            pltpu.VMEM_SHARED((NS, NPAD), jnp.float32),
            pltpu.VMEM_SHARED((NPAD,), jnp.float32),
            pltpu.SemaphoreType.DMA,
        ],
        compiler_params=_SC_PARAMS,
    )(ew, col)


def _y_body(x_ref, w1_ref, disbc_ref, y_ref):
    xw = jnp.dot(x_ref[...], w1_ref[...], preferred_element_type=jnp.float32)
    y_ref[...] = xw * disbc_ref[...]


def _scaled_xw(x_pad, W1, disbc):
    return pl.pallas_call(
        _y_body,
        grid=(NB,),
        in_specs=[
            pl.BlockSpec((BN, D), lambda i: (i, 0)),
            pl.BlockSpec((D, H), lambda i: (0, 0)),
            pl.BlockSpec((BN, D), lambda i: (i, 0)),
        ],
        out_specs=pl.BlockSpec((BN, H), lambda i: (i, 0)),
        out_shape=jax.ShapeDtypeStruct((NPAD, H), jnp.float32),
    )(x_pad, W1, disbc)


def _msg_body(y_hbm, row_hbm, col_hbm, ew_hbm, out_hbm,
              rowv, colv, colv2, ewv, rows, accsh, zsrc, sem):
    cid = lax.axis_index("c")
    sid = lax.axis_index("s")
    zero16 = jnp.zeros((L,), jnp.float32)
    nbase = cid * NH

    def zrow(i, _):
        for f in range(H // L):
            zsrc[i, pl.ds(f * L, L)] = zero16
        return 0
    lax.fori_loop(0, ZR, zrow, 0)
    pltpu.sync_copy(zsrc, accsh.at[pl.ds(sid * ZR, ZR)])
    plsc.subcore_barrier()

    def step(t, _):
        base = sid * ET1 + t * K
        pltpu.sync_copy(row_hbm.at[pl.ds(base, K)], rowv)
        pltpu.async_copy(y_hbm.at[rowv], rows, sem).wait()
        pltpu.sync_copy(ew_hbm.at[pl.ds(base, K)], ewv)
        pltpu.sync_copy(col_hbm.at[pl.ds(base, K)], colv)

        def grp(g, _):
            c16 = colv[pl.ds(g * L, L)] - nbase
            ok = (c16 >= 0) & (c16 < NH)
            colv2[pl.ds(g * L, L)] = jnp.where(ok, c16, NH)
            w16 = ewv[pl.ds(g * L, L)]
            for j in range(L):
                e = g * L + j
                w = w16[j]
                for f in range(H // L):
                    rows[e, pl.ds(f * L, L)] = rows[e, pl.ds(f * L, L)] * w
            return 0
        lax.fori_loop(0, K // L, grp, 0)
        pltpu.sync_copy(rows, accsh.at[colv2], add=True)
        return 0
    lax.fori_loop(0, ET1 // K, step, 0)
    plsc.subcore_barrier()

    cp = NH // 8

    @pl.when(sid < 8)
    def _():
        pltpu.sync_copy(accsh.at[pl.ds(sid * cp, cp)],
                        out_hbm.at[pl.ds(nbase + sid * cp, cp)])


def _msg_partials(y, row, col, ew):
    mesh = plsc.VectorSubcoreMesh(core_axis_name="c", subcore_axis_name="s",
                                  num_cores=NC, num_subcores=NS)
    return pl.kernel(
        _msg_body,
        out_type=jax.ShapeDtypeStruct((NPAD, H), jnp.float32),
        mesh=mesh,
        scratch_types=[
            pltpu.VMEM((K,), jnp.int32),
            pltpu.VMEM((K,), jnp.int32),
            pltpu.VMEM((K,), jnp.int32),
            pltpu.VMEM((K,), jnp.float32),
            pltpu.VMEM((K, H), jnp.float32),
            pltpu.VMEM_SHARED((AR, H), jnp.float32),
            pltpu.VMEM((ZR, H), jnp.float32),
            pltpu.SemaphoreType.DMA,
        ],
    )(y, row, col, ew)


def _final_body(disbc_ref, s_ref, y_ref, b1_ref, batch_ref,
                w2_ref, b2_ref, w3_ref, b3_ref, out_ref, pool, cnt):
    i = pl.program_id(0)

    @pl.when(i == 0)
    def _():
        pool[...] = jnp.zeros_like(pool)
        cnt[...] = jnp.zeros_like(cnt)

    dis = disbc_ref[...]
    x1 = jnp.maximum((s_ref[...] + y_ref[...]) * dis + b1_ref[...], 0.0)

    oh = (batch_ref[0, 0][:, None]
          == lax.broadcasted_iota(jnp.int32, (BN, G), 1)).astype(jnp.float32)
    pool[...] += jnp.dot(oh.T, x1, preferred_element_type=jnp.float32)
    cnt[...] += jnp.dot(oh.T, jnp.ones_like(x1),
                        preferred_element_type=jnp.float32)

    @pl.when(i == NB - 1)
    def _():
        pooled = pool[...] / jnp.maximum(cnt[...], 1.0)
        x2 = jnp.maximum(jnp.dot(pooled, w2_ref[...],
                                 preferred_element_type=jnp.float32)
                         + b2_ref[...], 0.0)
        out_ref[...] = jnp.dot(x2, w3_ref[...],
                               preferred_element_type=jnp.float32) + b3_ref[...]


def _final(disbc, s_part, y, b1, batch3, W2, b2, W3p, b3p):
    return pl.pallas_call(
        _final_body,
        grid=(NB,),
        in_specs=[
            pl.BlockSpec((BN, H), lambda i: (i, 0)),
            pl.BlockSpec((BN, H), lambda i: (i, 0)),
            pl.BlockSpec((BN, H), lambda i: (i, 0)),
            pl.BlockSpec((1, H), lambda i: (0, 0)),
            pl.BlockSpec((1, 1, BN), lambda i: (i, 0, 0)),
            pl.BlockSpec((H, H), lambda i: (0, 0)),
            pl.BlockSpec((1, H), lambda i: (0, 0)),
            pl.BlockSpec((H, H), lambda i: (0, 0)),
            pl.BlockSpec((1, H), lambda i: (0, 0)),
        ],
        out_specs=pl.BlockSpec((G, H), lambda i: (0, 0)),
        out_shape=jax.ShapeDtypeStruct((G, H), jnp.float32),
        scratch_shapes=[
            pltpu.VMEM((G, H), jnp.float32),
            pltpu.VMEM((G, H), jnp.float32),
        ],
    )(disbc, s_part, y, b1, batch3, W2, b2, W3p, b3p)


def kernel(x, edge_index, edge_attr, batch, W1, b1, W2, b2, W3, b3,
           We1, be1, We2, be2):
    row = edge_index[0]
    col = edge_index[1]
    rowf = row.astype(jnp.float32).reshape(E // 128, 128)
    colf = col.astype(jnp.float32).reshape(E // 128, 128)
    attrf = edge_attr.reshape(E // 128, 128)
    params = jnp.concatenate(
        [We1[:, 0], be1, We2[0], be2,
         jnp.zeros((122,), jnp.float32)]).reshape(1, 128)

    ew2d = _edge_weights(rowf, colf, attrf, params)
    ew = ew2d.reshape(E)

    disbc = _dis_broadcast(ew, col)

    x_pad = jnp.pad(x, ((0, NPAD - N), (0, 0)))
    y = _scaled_xw(x_pad, W1, disbc)
    s_part = _msg_partials(y, row, col, ew)

    batch3 = jnp.pad(batch, (0, NPAD - N),
                     constant_values=G).reshape(NB, 1, BN)
    W3p = jnp.pad(W3, ((0, 0), (0, H - A)))
    b3p = jnp.pad(b3, (0, H - A)).reshape(1, H)
    out = _final(disbc, s_part, y, b1.reshape(1, H), batch3,
                 W2, b2.reshape(1, H), W3p, b3p)
    return out[:, :A]


# speedup vs baseline: 16.6172x; 1.9594x over previous
import functools

import jax
import jax.numpy as jnp
from jax import lax
from jax.experimental import pallas as pl
from jax.experimental.pallas import tpu as pltpu
from jax.experimental.pallas import tpu_sc as plsc

N = 10000
E = 320000
D = 128
H = 128
A = 32
G = 16

NC = 2
NS = 16
L = 16

NPT = 640
NPAD = NS * NPT
ET1 = E // NS
KD = 2000
K = 80
NH = NPAD // NC
AR = 5248
ZR = AR // NS
DR = NH // NS

NB = 10
BN = NPAD // NB

_SC_PARAMS = pltpu.CompilerParams(needs_layout_passes=False)


def _ew_body(rowf_ref, colf_ref, attr_ref, p_ref, out_ref):
    p = p_ref[0]
    h = jnp.maximum(rowf_ref[...] * p[0] + colf_ref[...] * p[1]
                    + attr_ref[...] * p[2] + p[3], 0.0)
    out_ref[...] = jax.nn.sigmoid(h * p[4] + p[5])


def _edge_weights(rowf, colf, attrf, params):
    return pl.pallas_call(
        _ew_body,
        out_shape=jax.ShapeDtypeStruct((E // 128, 128), jnp.float32),
    )(rowf, colf, attrf, params)


def _deg_body(ew_hbm, col_hbm, disbc_hbm,
              colv, ewv, acc, tmpv, dredv, drv, dbuf, redsh, dissh, sem):
    cid = lax.axis_index("c")
    sid = lax.axis_index("s")
    zero16 = jnp.zeros((L,), jnp.float32)

    def zg(g, _):
        acc[pl.ds(g * L, L)] = zero16
        return 0
    lax.fori_loop(0, NPAD // L, zg, 0)

    def step(t, _):
        base = sid * ET1 + t * KD
        pltpu.sync_copy(ew_hbm.at[pl.ds(base, KD)], ewv)
        pltpu.sync_copy(col_hbm.at[pl.ds(base, KD)], colv)

        def grp(g, _):
            c16 = colv[pl.ds(g * L, L)]
            w16 = ewv[pl.ds(g * L, L)]
            plsc.addupdate_scatter(acc, [c16], w16)
            return 0
        lax.fori_loop(0, KD // L, grp, 0)
        return 0
    lax.fori_loop(0, ET1 // KD, step, 0)

    pltpu.sync_copy(acc, redsh.at[sid])
    plsc.subcore_barrier()

    def zg2(g, _):
        dredv[pl.ds(g * L, L)] = zero16
        return 0
    lax.fori_loop(0, NPT // L, zg2, 0)
    for t in range(NS):
        pltpu.sync_copy(redsh.at[t, pl.ds(sid * NPT, NPT)], tmpv)

        def addg(g, _):
            dredv[pl.ds(g * L, L)] = (dredv[pl.ds(g * L, L)]
                                      + tmpv[pl.ds(g * L, L)])
            return 0
        lax.fori_loop(0, NPT // L, addg, 0)

    def newt(g, _):
        d = 1.0 + dredv[pl.ds(g * L, L)]
        bits = plsc.bitcast(d, jnp.int32)
        yv = plsc.bitcast(jnp.int32(0x5F3759DF)
                          - lax.shift_right_logical(bits, 1), jnp.float32)
        for _ in range(3):
            yv = yv * (1.5 - 0.5 * d * yv * yv)
        dredv[pl.ds(g * L, L)] = yv
        return 0
    lax.fori_loop(0, NPT // L, newt, 0)

    pltpu.sync_copy(dredv, dissh.at[pl.ds(sid * NPT, NPT)])
    plsc.subcore_barrier()

    obase = cid * NH + sid * DR
    pltpu.sync_copy(dissh.at[pl.ds(obase, DR)], drv)

    def brow(g, _):
        d16 = drv[pl.ds(g * L, L)]
        for j in range(L):
            r = g * L + j
            dj = jnp.full((L,), d16[j], jnp.float32)
            for f in range(D // L):
                dbuf[r, pl.ds(f * L, L)] = dj
        return 0
    lax.fori_loop(0, DR // L, brow, 0)
    pltpu.sync_copy(dbuf, disbc_hbm.at[pl.ds(obase, DR)])


def _dis_broadcast(ew, col):
    mesh = plsc.VectorSubcoreMesh(core_axis_name="c", subcore_axis_name="s",
                                  num_cores=NC, num_subcores=NS)
    return pl.kernel(
        _deg_body,
        out_type=jax.ShapeDtypeStruct((NPAD, D), jnp.float32),
        mesh=mesh,
        scratch_types=[
            pltpu.VMEM((KD,), jnp.int32),
            pltpu.VMEM((KD,), jnp.float32),
            pltpu.VMEM((NPAD,), jnp.float32),
            pltpu.VMEM((NPT,), jnp.float32),
            pltpu.VMEM((NPT,), jnp.float32),
            pltpu.VMEM((DR,), jnp.float32),
            pltpu.VMEM((DR, D), jnp.float32),
            pltpu.VMEM_SHARED((NS, NPAD), jnp.float32),
            pltpu.VMEM_SHARED((NPAD,), jnp.float32),
            pltpu.SemaphoreType.DMA,
        ],
        compiler_params=_SC_PARAMS,
    )(ew, col)


def _y_body(x_ref, w1_ref, disbc_ref, y_ref):
    xw = jnp.dot(x_ref[...], w1_ref[...], preferred_element_type=jnp.float32)
    y_ref[...] = xw * disbc_ref[...]


def _scaled_xw(x_pad, W1, disbc):
    return pl.pallas_call(
        _y_body,
        grid=(NB,),
        in_specs=[
            pl.BlockSpec((BN, D), lambda i: (i, 0)),
            pl.BlockSpec((D, H), lambda i: (0, 0)),
            pl.BlockSpec((BN, D), lambda i: (i, 0)),
        ],
        out_specs=pl.BlockSpec((BN, H), lambda i: (i, 0)),
        out_shape=jax.ShapeDtypeStruct((NPAD, H), jnp.float32),
    )(x_pad, W1, disbc)


NCH = ET1 // K
DEPTH = 3


def _msg_body(y_hbm, row_hbm, col_hbm, ew_hbm, out_hbm,
              rowv_all, colv, ewv,
              rows0, rows1, rows2, cv0, cv1, cv2, zbuf, accsh,
              gs0, gs1, gs2, ss0, ss1, ss2):
    cid = lax.axis_index("c")
    sid = lax.axis_index("s")
    zero16 = jnp.zeros((L,), jnp.float32)
    nbase = cid * NH
    ebase = sid * ET1

    rows_b = (rows0, rows1, rows2)
    cv_b = (cv0, cv1, cv2)
    gs_b = (gs0, gs1, gs2)
    ss_b = (ss0, ss1, ss2)

    for r in range(8):
        for f in range(H // L):
            zbuf[r, pl.ds(f * L, L)] = zero16

    def zcp(i, _):
        pltpu.sync_copy(zbuf, accsh.at[pl.ds(sid * ZR + i * 8, 8)])
        return 0
    lax.fori_loop(0, ZR // 8, zcp, 0)

    pltpu.sync_copy(row_hbm.at[pl.ds(ebase, ET1)], rowv_all)
    plsc.subcore_barrier()

    def issue_g(c, b):
        pltpu.async_copy(y_hbm.at[rowv_all.at[pl.ds(c * K, K)]],
                         rows_b[b], gs_b[b])

    def chunk(c, b):
        pltpu.make_async_copy(y_hbm.at[rowv_all.at[pl.ds(c * K, K)]],
                              rows_b[b], gs_b[b]).wait()
        cb = ebase + c * K
        pltpu.sync_copy(col_hbm.at[pl.ds(cb, K)], colv)
        pltpu.sync_copy(ew_hbm.at[pl.ds(cb, K)], ewv)

        def grp(g, _):
            c16 = colv[pl.ds(g * L, L)] - nbase
            ok = (c16 >= 0) & (c16 < NH)
            cv_b[b][pl.ds(g * L, L)] = jnp.where(ok, c16, NH)
            w16 = ewv[pl.ds(g * L, L)]
            for j in range(L):
                e = g * L + j
                w = w16[j]
                for f in range(H // L):
                    rows_b[b][e, pl.ds(f * L, L)] = (
                        rows_b[b][e, pl.ds(f * L, L)] * w)
            return 0
        lax.fori_loop(0, K // L, grp, 0)
        pltpu.async_copy(rows_b[b], accsh.at[cv_b[b]], ss_b[b], add=True)

        bm = (b + 2) % DEPTH

        @pl.when(c >= 1)
        def _():
            pltpu.make_async_copy(rows_b[bm], accsh.at[cv_b[bm]],
                                  ss_b[bm]).wait()

        @pl.when(c + 2 < NCH)
        def _():
            issue_g(c + 2, bm)

    issue_g(0, 0)
    issue_g(1, 1)

    def main(t, _):
        c0 = t * DEPTH
        for off in range(DEPTH):
            chunk(c0 + off, off)
        return 0
    lax.fori_loop(0, (NCH - 1) // DEPTH, main, 0)
    chunk(NCH - 1, (NCH - 1) % DEPTH)
    pltpu.make_async_copy(rows_b[(NCH - 1) % DEPTH],
                          accsh.at[cv_b[(NCH - 1) % DEPTH]],
                          ss_b[(NCH - 1) % DEPTH]).wait()

    plsc.subcore_barrier()

    cp = NH // 8

    @pl.when(sid < 8)
    def _():
        pltpu.sync_copy(accsh.at[pl.ds(sid * cp, cp)],
                        out_hbm.at[pl.ds(nbase + sid * cp, cp)])


def _msg_partials(y, row, col, ew):
    mesh = plsc.VectorSubcoreMesh(core_axis_name="c", subcore_axis_name="s",
                                  num_cores=NC, num_subcores=NS)
    return pl.kernel(
        _msg_body,
        out_type=jax.ShapeDtypeStruct((NPAD, H), jnp.float32),
        mesh=mesh,
        scratch_types=[
            pltpu.VMEM((ET1,), jnp.int32),
            pltpu.VMEM((K,), jnp.int32),
            pltpu.VMEM((K,), jnp.float32),
            pltpu.VMEM((K, H), jnp.float32),
            pltpu.VMEM((K, H), jnp.float32),
            pltpu.VMEM((K, H), jnp.float32),
            pltpu.VMEM((K,), jnp.int32),
            pltpu.VMEM((K,), jnp.int32),
            pltpu.VMEM((K,), jnp.int32),
            pltpu.VMEM((8, H), jnp.float32),
            pltpu.VMEM_SHARED((AR, H), jnp.float32),
            pltpu.SemaphoreType.DMA,
            pltpu.SemaphoreType.DMA,
            pltpu.SemaphoreType.DMA,
            pltpu.SemaphoreType.DMA,
            pltpu.SemaphoreType.DMA,
            pltpu.SemaphoreType.DMA,
        ],
    )(y, row, col, ew)


def _final_body(disbc_ref, s_ref, y_ref, b1_ref, batch_ref,
                w2_ref, b2_ref, w3_ref, b3_ref, out_ref, pool, cnt):
    i = pl.program_id(0)

    @pl.when(i == 0)
    def _():
        pool[...] = jnp.zeros_like(pool)
        cnt[...] = jnp.zeros_like(cnt)

    dis = disbc_ref[...]
    x1 = jnp.maximum((s_ref[...] + y_ref[...]) * dis + b1_ref[...], 0.0)

    oh = (batch_ref[0, 0][:, None]
          == lax.broadcasted_iota(jnp.int32, (BN, G), 1)).astype(jnp.float32)
    pool[...] += jnp.dot(oh.T, x1, preferred_element_type=jnp.float32)
    cnt[...] += jnp.dot(oh.T, jnp.ones_like(x1),
                        preferred_element_type=jnp.float32)

    @pl.when(i == NB - 1)
    def _():
        pooled = pool[...] / jnp.maximum(cnt[...], 1.0)
        x2 = jnp.maximum(jnp.dot(pooled, w2_ref[...],
                                 preferred_element_type=jnp.float32)
                         + b2_ref[...], 0.0)
        out_ref[...] = jnp.dot(x2, w3_ref[...],
                               preferred_element_type=jnp.float32) + b3_ref[...]


def _final(disbc, s_part, y, b1, batch3, W2, b2, W3p, b3p):
    return pl.pallas_call(
        _final_body,
        grid=(NB,),
        in_specs=[
            pl.BlockSpec((BN, H), lambda i: (i, 0)),
            pl.BlockSpec((BN, H), lambda i: (i, 0)),
            pl.BlockSpec((BN, H), lambda i: (i, 0)),
            pl.BlockSpec((1, H), lambda i: (0, 0)),
            pl.BlockSpec((1, 1, BN), lambda i: (i, 0, 0)),
            pl.BlockSpec((H, H), lambda i: (0, 0)),
            pl.BlockSpec((1, H), lambda i: (0, 0)),
            pl.BlockSpec((H, H), lambda i: (0, 0)),
            pl.BlockSpec((1, H), lambda i: (0, 0)),
        ],
        out_specs=pl.BlockSpec((G, H), lambda i: (0, 0)),
        out_shape=jax.ShapeDtypeStruct((G, H), jnp.float32),
        scratch_shapes=[
            pltpu.VMEM((G, H), jnp.float32),
            pltpu.VMEM((G, H), jnp.float32),
        ],
    )(disbc, s_part, y, b1, batch3, W2, b2, W3p, b3p)


def kernel(x, edge_index, edge_attr, batch, W1, b1, W2, b2, W3, b3,
           We1, be1, We2, be2):
    row = edge_index[0]
    col = edge_index[1]
    rowf = row.astype(jnp.float32).reshape(E // 128, 128)
    colf = col.astype(jnp.float32).reshape(E // 128, 128)
    attrf = edge_attr.reshape(E // 128, 128)
    params = jnp.concatenate(
        [We1[:, 0], be1, We2[0], be2,
         jnp.zeros((122,), jnp.float32)]).reshape(1, 128)

    ew2d = _edge_weights(rowf, colf, attrf, params)
    ew = ew2d.reshape(E)

    disbc = _dis_broadcast(ew, col)

    x_pad = jnp.pad(x, ((0, NPAD - N), (0, 0)))
    y = _scaled_xw(x_pad, W1, disbc)
    s_part = _msg_partials(y, row, col, ew)

    batch3 = jnp.pad(batch, (0, NPAD - N),
                     constant_values=G).reshape(NB, 1, BN)
    W3p = jnp.pad(W3, ((0, 0), (0, H - A)))
    b3p = jnp.pad(b3, (0, H - A)).reshape(1, H)
    out = _final(disbc, s_part, y, b1.reshape(1, H), batch3,
                 W2, b2.reshape(1, H), W3p, b3p)
    return out[:, :A]


# speedup vs baseline: 19.0846x; 1.1485x over previous
import functools

import jax
import jax.numpy as jnp
from jax import lax
from jax.experimental import pallas as pl
from jax.experimental.pallas import tpu as pltpu
from jax.experimental.pallas import tpu_sc as plsc

N = 10000
E = 320000
D = 128
H = 128
A = 32
G = 16

NC = 2
NS = 16
L = 16

NPT = 640
NPAD = NS * NPT
ET1 = E // NS
KD = 2000
K = 80
NH = NPAD // NC
AR = 5248
ZR = AR // NS
DR = NH // NS

NB = 10
BN = NPAD // NB

_SC_PARAMS = pltpu.CompilerParams(needs_layout_passes=False)


def _ew_body(rowf_ref, colf_ref, attr_ref, p_ref, out_ref):
    p = p_ref[0]
    h = jnp.maximum(rowf_ref[...] * p[0] + colf_ref[...] * p[1]
                    + attr_ref[...] * p[2] + p[3], 0.0)
    out_ref[...] = jax.nn.sigmoid(h * p[4] + p[5])


def _edge_weights(rowf, colf, attrf, params):
    return pl.pallas_call(
        _ew_body,
        out_shape=jax.ShapeDtypeStruct((E // 128, 128), jnp.float32),
    )(rowf, colf, attrf, params)


def _deg_body(ew_hbm, col_hbm, disbc_hbm,
              colv, ewv, acc, tmpv, dredv, drv, dbuf, redsh, dissh, sem):
    cid = lax.axis_index("c")
    sid = lax.axis_index("s")
    zero16 = jnp.zeros((L,), jnp.float32)

    def zg(g, _):
        acc[pl.ds(g * L, L)] = zero16
        return 0
    lax.fori_loop(0, NPAD // L, zg, 0)

    def step(t, _):
        base = sid * ET1 + t * KD
        pltpu.sync_copy(ew_hbm.at[pl.ds(base, KD)], ewv)
        pltpu.sync_copy(col_hbm.at[pl.ds(base, KD)], colv)

        def grp(g, _):
            c16 = colv[pl.ds(g * L, L)]
            w16 = ewv[pl.ds(g * L, L)]
            plsc.addupdate_scatter(acc, [c16], w16)
            return 0
        lax.fori_loop(0, KD // L, grp, 0)
        return 0
    lax.fori_loop(0, ET1 // KD, step, 0)

    pltpu.sync_copy(acc, redsh.at[sid])
    plsc.subcore_barrier()

    def zg2(g, _):
        dredv[pl.ds(g * L, L)] = zero16
        return 0
    lax.fori_loop(0, NPT // L, zg2, 0)
    for t in range(NS):
        pltpu.sync_copy(redsh.at[t, pl.ds(sid * NPT, NPT)], tmpv)

        def addg(g, _):
            dredv[pl.ds(g * L, L)] = (dredv[pl.ds(g * L, L)]
                                      + tmpv[pl.ds(g * L, L)])
            return 0
        lax.fori_loop(0, NPT // L, addg, 0)

    def newt(g, _):
        d = 1.0 + dredv[pl.ds(g * L, L)]
        bits = plsc.bitcast(d, jnp.int32)
        yv = plsc.bitcast(jnp.int32(0x5F3759DF)
                          - lax.shift_right_logical(bits, 1), jnp.float32)
        for _ in range(3):
            yv = yv * (1.5 - 0.5 * d * yv * yv)
        dredv[pl.ds(g * L, L)] = yv
        return 0
    lax.fori_loop(0, NPT // L, newt, 0)

    pltpu.sync_copy(dredv, dissh.at[pl.ds(sid * NPT, NPT)])
    plsc.subcore_barrier()

    obase = cid * NH + sid * DR
    pltpu.sync_copy(dissh.at[pl.ds(obase, DR)], drv)

    def brow(g, _):
        d16 = drv[pl.ds(g * L, L)]
        for j in range(L):
            r = g * L + j
            dj = jnp.full((L,), d16[j], jnp.float32)
            for f in range(D // L):
                dbuf[r, pl.ds(f * L, L)] = dj
        return 0
    lax.fori_loop(0, DR // L, brow, 0)
    pltpu.sync_copy(dbuf, disbc_hbm.at[pl.ds(obase, DR)])


def _dis_broadcast(ew, col):
    mesh = plsc.VectorSubcoreMesh(core_axis_name="c", subcore_axis_name="s",
                                  num_cores=NC, num_subcores=NS)
    return pl.kernel(
        _deg_body,
        out_type=jax.ShapeDtypeStruct((NPAD, D), jnp.float32),
        mesh=mesh,
        scratch_types=[
            pltpu.VMEM((KD,), jnp.int32),
            pltpu.VMEM((KD,), jnp.float32),
            pltpu.VMEM((NPAD,), jnp.float32),
            pltpu.VMEM((NPT,), jnp.float32),
            pltpu.VMEM((NPT,), jnp.float32),
            pltpu.VMEM((DR,), jnp.float32),
            pltpu.VMEM((DR, D), jnp.float32),
            pltpu.VMEM_SHARED((NS, NPAD), jnp.float32),
            pltpu.VMEM_SHARED((NPAD,), jnp.float32),
            pltpu.SemaphoreType.DMA,
        ],
        compiler_params=_SC_PARAMS,
    )(ew, col)


def _y_body(x_ref, w1_ref, disbc_ref, y_ref):
    xw = jnp.dot(x_ref[...], w1_ref[...], preferred_element_type=jnp.float32)
    y_ref[...] = xw * disbc_ref[...]


def _scaled_xw(x_pad, W1, disbc):
    return pl.pallas_call(
        _y_body,
        grid=(NB,),
        in_specs=[
            pl.BlockSpec((BN, D), lambda i: (i, 0)),
            pl.BlockSpec((D, H), lambda i: (0, 0)),
            pl.BlockSpec((BN, D), lambda i: (i, 0)),
        ],
        out_specs=pl.BlockSpec((BN, H), lambda i: (i, 0)),
        out_shape=jax.ShapeDtypeStruct((NPAD, H), jnp.float32),
    )(x_pad, W1, disbc)


NCH = ET1 // K
DEPTH = 3


def _msg_body(y_hbm, row_hbm, col_hbm, ew_hbm, out_hbm,
              rowv_all, cl0, cl1, cl2, el0, el1, el2,
              rows0, rows1, rows2, cv0, cv1, cv2, zbuf, accsh,
              gs0, gs1, gs2, ss0, ss1, ss2, es0, es1, es2):
    cid = lax.axis_index("c")
    sid = lax.axis_index("s")
    zero16 = jnp.zeros((L,), jnp.float32)
    nbase = cid * NH
    ebase = sid * ET1

    rows_b = (rows0, rows1, rows2)
    cv_b = (cv0, cv1, cv2)
    cl_b = (cl0, cl1, cl2)
    el_b = (el0, el1, el2)
    gs_b = (gs0, gs1, gs2)
    ss_b = (ss0, ss1, ss2)
    es_b = (es0, es1, es2)

    for r in range(8):
        for f in range(H // L):
            zbuf[r, pl.ds(f * L, L)] = zero16

    def zcp(i, _):
        pltpu.sync_copy(zbuf, accsh.at[pl.ds(sid * ZR + i * 8, 8)])
        return 0
    lax.fori_loop(0, ZR // 8, zcp, 0)

    pltpu.sync_copy(row_hbm.at[pl.ds(ebase, ET1)], rowv_all)
    plsc.subcore_barrier()

    def issue_g(c, b):
        pltpu.async_copy(y_hbm.at[rowv_all.at[pl.ds(c * K, K)]],
                         rows_b[b], gs_b[b])
        pltpu.async_copy(col_hbm.at[pl.ds(ebase + c * K, K)], cl_b[b], es_b[b])
        pltpu.async_copy(ew_hbm.at[pl.ds(ebase + c * K, K)], el_b[b], es_b[b])

    def chunk(c, b):
        pltpu.make_async_copy(y_hbm.at[rowv_all.at[pl.ds(c * K, K)]],
                              rows_b[b], gs_b[b]).wait()
        cb = ebase + c * K
        pltpu.make_async_copy(col_hbm.at[pl.ds(cb, K)], cl_b[b],
                              es_b[b]).wait()
        pltpu.make_async_copy(ew_hbm.at[pl.ds(cb, K)], el_b[b],
                              es_b[b]).wait()

        def grp(g, _):
            c16 = cl_b[b][pl.ds(g * L, L)] - nbase
            ok = (c16 >= 0) & (c16 < NH)
            cv_b[b][pl.ds(g * L, L)] = jnp.where(ok, c16, NH)
            w16 = el_b[b][pl.ds(g * L, L)]
            for j in range(L):
                e = g * L + j
                w = w16[j]
                for f in range(H // L):
                    rows_b[b][e, pl.ds(f * L, L)] = (
                        rows_b[b][e, pl.ds(f * L, L)] * w)
            return 0
        lax.fori_loop(0, K // L, grp, 0)
        pltpu.async_copy(rows_b[b], accsh.at[cv_b[b]], ss_b[b], add=True)

        bm = (b + 2) % DEPTH

        @pl.when(c >= 1)
        def _():
            pltpu.make_async_copy(rows_b[bm], accsh.at[cv_b[bm]],
                                  ss_b[bm]).wait()

        @pl.when(c + 2 < NCH)
        def _():
            issue_g(c + 2, bm)

    issue_g(0, 0)
    issue_g(1, 1)

    def main(t, _):
        c0 = t * DEPTH
        for off in range(DEPTH):
            chunk(c0 + off, off)
        return 0
    lax.fori_loop(0, (NCH - 1) // DEPTH, main, 0)
    chunk(NCH - 1, (NCH - 1) % DEPTH)
    pltpu.make_async_copy(rows_b[(NCH - 1) % DEPTH],
                          accsh.at[cv_b[(NCH - 1) % DEPTH]],
                          ss_b[(NCH - 1) % DEPTH]).wait()

    plsc.subcore_barrier()

    cp = NH // 8

    @pl.when(sid < 8)
    def _():
        pltpu.sync_copy(accsh.at[pl.ds(sid * cp, cp)],
                        out_hbm.at[pl.ds(nbase + sid * cp, cp)])


def _msg_partials(y, row, col, ew):
    mesh = plsc.VectorSubcoreMesh(core_axis_name="c", subcore_axis_name="s",
                                  num_cores=NC, num_subcores=NS)
    return pl.kernel(
        _msg_body,
        out_type=jax.ShapeDtypeStruct((NPAD, H), jnp.float32),
        mesh=mesh,
        scratch_types=[
            pltpu.VMEM((ET1,), jnp.int32),
            pltpu.VMEM((K,), jnp.int32),
            pltpu.VMEM((K,), jnp.int32),
            pltpu.VMEM((K,), jnp.int32),
            pltpu.VMEM((K,), jnp.float32),
            pltpu.VMEM((K,), jnp.float32),
            pltpu.VMEM((K,), jnp.float32),
            pltpu.VMEM((K, H), jnp.float32),
            pltpu.VMEM((K, H), jnp.float32),
            pltpu.VMEM((K, H), jnp.float32),
            pltpu.VMEM((K,), jnp.int32),
            pltpu.VMEM((K,), jnp.int32),
            pltpu.VMEM((K,), jnp.int32),
            pltpu.VMEM((8, H), jnp.float32),
            pltpu.VMEM_SHARED((AR, H), jnp.float32),
            pltpu.SemaphoreType.DMA,
            pltpu.SemaphoreType.DMA,
            pltpu.SemaphoreType.DMA,
            pltpu.SemaphoreType.DMA,
            pltpu.SemaphoreType.DMA,
            pltpu.SemaphoreType.DMA,
            pltpu.SemaphoreType.DMA,
            pltpu.SemaphoreType.DMA,
            pltpu.SemaphoreType.DMA,
        ],
    )(y, row, col, ew)


def _final_body(disbc_ref, s_ref, y_ref, b1_ref, batch_ref,
                w2_ref, b2_ref, w3_ref, b3_ref, out_ref, pool, cnt):
    i = pl.program_id(0)

    @pl.when(i == 0)
    def _():
        pool[...] = jnp.zeros_like(pool)
        cnt[...] = jnp.zeros_like(cnt)

    dis = disbc_ref[...]
    x1 = jnp.maximum((s_ref[...] + y_ref[...]) * dis + b1_ref[...], 0.0)

    oh = (batch_ref[0, 0][:, None]
          == lax.broadcasted_iota(jnp.int32, (BN, G), 1)).astype(jnp.float32)
    pool[...] += jnp.dot(oh.T, x1, preferred_element_type=jnp.float32)
    cnt[...] += jnp.dot(oh.T, jnp.ones_like(x1),
                        preferred_element_type=jnp.float32)

    @pl.when(i == NB - 1)
    def _():
        pooled = pool[...] / jnp.maximum(cnt[...], 1.0)
        x2 = jnp.maximum(jnp.dot(pooled, w2_ref[...],
                                 preferred_element_type=jnp.float32)
                         + b2_ref[...], 0.0)
        out_ref[...] = jnp.dot(x2, w3_ref[...],
                               preferred_element_type=jnp.float32) + b3_ref[...]


def _final(disbc, s_part, y, b1, batch3, W2, b2, W3p, b3p):
    return pl.pallas_call(
        _final_body,
        grid=(NB,),
        in_specs=[
            pl.BlockSpec((BN, H), lambda i: (i, 0)),
            pl.BlockSpec((BN, H), lambda i: (i, 0)),
            pl.BlockSpec((BN, H), lambda i: (i, 0)),
            pl.BlockSpec((1, H), lambda i: (0, 0)),
            pl.BlockSpec((1, 1, BN), lambda i: (i, 0, 0)),
            pl.BlockSpec((H, H), lambda i: (0, 0)),
            pl.BlockSpec((1, H), lambda i: (0, 0)),
            pl.BlockSpec((H, H), lambda i: (0, 0)),
            pl.BlockSpec((1, H), lambda i: (0, 0)),
        ],
        out_specs=pl.BlockSpec((G, H), lambda i: (0, 0)),
        out_shape=jax.ShapeDtypeStruct((G, H), jnp.float32),
        scratch_shapes=[
            pltpu.VMEM((G, H), jnp.float32),
            pltpu.VMEM((G, H), jnp.float32),
        ],
    )(disbc, s_part, y, b1, batch3, W2, b2, W3p, b3p)


def kernel(x, edge_index, edge_attr, batch, W1, b1, W2, b2, W3, b3,
           We1, be1, We2, be2):
    row = edge_index[0]
    col = edge_index[1]
    rowf = row.astype(jnp.float32).reshape(E // 128, 128)
    colf = col.astype(jnp.float32).reshape(E // 128, 128)
    attrf = edge_attr.reshape(E // 128, 128)
    params = jnp.concatenate(
        [We1[:, 0], be1, We2[0], be2,
         jnp.zeros((122,), jnp.float32)]).reshape(1, 128)

    ew2d = _edge_weights(rowf, colf, attrf, params)
    ew = ew2d.reshape(E)

    disbc = _dis_broadcast(ew, col)

    x_pad = jnp.pad(x, ((0, NPAD - N), (0, 0)))
    y = _scaled_xw(x_pad, W1, disbc)
    s_part = _msg_partials(y, row, col, ew)

    batch3 = jnp.pad(batch, (0, NPAD - N),
                     constant_values=G).reshape(NB, 1, BN)
    W3p = jnp.pad(W3, ((0, 0), (0, H - A)))
    b3p = jnp.pad(b3, (0, H - A)).reshape(1, H)
    out = _final(disbc, s_part, y, b1.reshape(1, H), batch3,
                 W2, b2.reshape(1, H), W3p, b3p)
    return out[:, :A]


# speedup vs baseline: 19.3614x; 1.0145x over previous
import functools

import jax
import jax.numpy as jnp
from jax import lax
from jax.experimental import pallas as pl
from jax.experimental.pallas import tpu as pltpu
from jax.experimental.pallas import tpu_sc as plsc

N = 10000
E = 320000
D = 128
H = 128
A = 32
G = 16

NC = 2
NS = 16
L = 16

NPT = 640
NPAD = NS * NPT
ET1 = E // NS
KD = 2000
K = 80
NH = NPAD // NC
AR = 5248
ZR = AR // NS
DR = NH // NS

NB = 10
BN = NPAD // NB

_SC_PARAMS = pltpu.CompilerParams(needs_layout_passes=False)


def _ew_body(rowf_ref, colf_ref, attr_ref, p_ref, x_ref, w1_ref,
             out_ref, xw_ref):
    p = p_ref[0]
    h = jnp.maximum(rowf_ref[...] * p[0] + colf_ref[...] * p[1]
                    + attr_ref[...] * p[2] + p[3], 0.0)
    out_ref[...] = jax.nn.sigmoid(h * p[4] + p[5])
    xw_ref[...] = jnp.dot(x_ref[...], w1_ref[...],
                          preferred_element_type=jnp.float32)


def _edge_weights_xw(rowf, colf, attrf, params, x_pad, W1):
    return pl.pallas_call(
        _ew_body,
        out_shape=[jax.ShapeDtypeStruct((E // 128, 128), jnp.float32),
                   jax.ShapeDtypeStruct((NPAD, H), jnp.float32)],
    )(rowf, colf, attrf, params, x_pad, W1)


def _deg_body(ew_hbm, col_hbm, disbc_hbm, disf_hbm,
              colv, ewv, acc, tmpv, dredv, drv, dbuf, redsh, dissh, sem):
    cid = lax.axis_index("c")
    sid = lax.axis_index("s")
    zero16 = jnp.zeros((L,), jnp.float32)

    def zg(g, _):
        acc[pl.ds(g * L, L)] = zero16
        return 0
    lax.fori_loop(0, NPAD // L, zg, 0)

    pltpu.sync_copy(ew_hbm.at[pl.ds(sid * ET1, ET1)], ewv)
    pltpu.sync_copy(col_hbm.at[pl.ds(sid * ET1, ET1)], colv)

    def grp(g, _):
        c16 = colv[pl.ds(g * L, L)]
        w16 = ewv[pl.ds(g * L, L)]
        plsc.addupdate_scatter(acc, [c16], w16)
        return 0
    lax.fori_loop(0, ET1 // L, grp, 0)

    pltpu.sync_copy(acc, redsh.at[sid])
    plsc.subcore_barrier()

    def zg2(g, _):
        dredv[pl.ds(g * L, L)] = zero16
        return 0
    lax.fori_loop(0, NPT // L, zg2, 0)
    for t in range(NS):
        pltpu.sync_copy(redsh.at[t, pl.ds(sid * NPT, NPT)], tmpv)

        def addg(g, _):
            dredv[pl.ds(g * L, L)] = (dredv[pl.ds(g * L, L)]
                                      + tmpv[pl.ds(g * L, L)])
            return 0
        lax.fori_loop(0, NPT // L, addg, 0)

    def newt(g, _):
        d = 1.0 + dredv[pl.ds(g * L, L)]
        bits = plsc.bitcast(d, jnp.int32)
        yv = plsc.bitcast(jnp.int32(0x5F3759DF)
                          - lax.shift_right_logical(bits, 1), jnp.float32)
        for _ in range(3):
            yv = yv * (1.5 - 0.5 * d * yv * yv)
        dredv[pl.ds(g * L, L)] = yv
        return 0
    lax.fori_loop(0, NPT // L, newt, 0)

    pltpu.sync_copy(dredv, dissh.at[pl.ds(sid * NPT, NPT)])

    @pl.when(cid == 0)
    def _():
        pltpu.sync_copy(dredv, disf_hbm.at[pl.ds(sid * NPT, NPT)])
    plsc.subcore_barrier()

    obase = cid * NH + sid * DR
    pltpu.sync_copy(dissh.at[pl.ds(obase, DR)], drv)

    def brow(g, _):
        d16 = drv[pl.ds(g * L, L)]
        for j in range(L):
            r = g * L + j
            dj = jnp.full((L,), d16[j], jnp.float32)
            for f in range(D // L):
                dbuf[r, pl.ds(f * L, L)] = dj
        return 0
    lax.fori_loop(0, DR // L, brow, 0)
    pltpu.sync_copy(dbuf, disbc_hbm.at[pl.ds(obase, DR)])


def _dis_broadcast(ew, col):
    mesh = plsc.VectorSubcoreMesh(core_axis_name="c", subcore_axis_name="s",
                                  num_cores=NC, num_subcores=NS)
    return pl.kernel(
        _deg_body,
        out_type=[jax.ShapeDtypeStruct((NPAD, D), jnp.float32),
                  jax.ShapeDtypeStruct((NPAD,), jnp.float32)],
        mesh=mesh,
        scratch_types=[
            pltpu.VMEM((ET1,), jnp.int32),
            pltpu.VMEM((ET1,), jnp.float32),
            pltpu.VMEM((NPAD,), jnp.float32),
            pltpu.VMEM((NPT,), jnp.float32),
            pltpu.VMEM((NPT,), jnp.float32),
            pltpu.VMEM((DR,), jnp.float32),
            pltpu.VMEM((DR, D), jnp.float32),
            pltpu.VMEM_SHARED((NS, NPAD), jnp.float32),
            pltpu.VMEM_SHARED((NPAD,), jnp.float32),
            pltpu.SemaphoreType.DMA,
        ],
        compiler_params=_SC_PARAMS,
    )(ew, col)


NCH = ET1 // K
DEPTH = 3


def _msg_body(y_hbm, disf_hbm, row_hbm, col_hbm, ew_hbm, out_hbm,
              rowv_all, disv, cl0, cl1, cl2, el0, el1, el2,
              rows0, rows1, rows2, cv0, cv1, cv2, zbuf, accsh,
              gs0, gs1, gs2, ss0, ss1, ss2, es0, es1, es2):
    cid = lax.axis_index("c")
    sid = lax.axis_index("s")
    zero16 = jnp.zeros((L,), jnp.float32)
    nbase = cid * NH
    ebase = sid * ET1

    rows_b = (rows0, rows1, rows2)
    cv_b = (cv0, cv1, cv2)
    cl_b = (cl0, cl1, cl2)
    el_b = (el0, el1, el2)
    gs_b = (gs0, gs1, gs2)
    ss_b = (ss0, ss1, ss2)
    es_b = (es0, es1, es2)

    for r in range(8):
        for f in range(H // L):
            zbuf[r, pl.ds(f * L, L)] = zero16

    def zcp(i, _):
        pltpu.sync_copy(zbuf, accsh.at[pl.ds(sid * ZR + i * 8, 8)])
        return 0
    lax.fori_loop(0, ZR // 8, zcp, 0)

    pltpu.sync_copy(row_hbm.at[pl.ds(ebase, ET1)], rowv_all)
    pltpu.sync_copy(disf_hbm, disv)
    plsc.subcore_barrier()

    def issue_g(c, b):
        pltpu.async_copy(y_hbm.at[rowv_all.at[pl.ds(c * K, K)]],
                         rows_b[b], gs_b[b])
        pltpu.async_copy(col_hbm.at[pl.ds(ebase + c * K, K)], cl_b[b], es_b[b])
        pltpu.async_copy(ew_hbm.at[pl.ds(ebase + c * K, K)], el_b[b], es_b[b])

    def chunk(c, b):
        pltpu.make_async_copy(y_hbm.at[rowv_all.at[pl.ds(c * K, K)]],
                              rows_b[b], gs_b[b]).wait()
        cb = ebase + c * K
        pltpu.make_async_copy(col_hbm.at[pl.ds(cb, K)], cl_b[b],
                              es_b[b]).wait()
        pltpu.make_async_copy(ew_hbm.at[pl.ds(cb, K)], el_b[b],
                              es_b[b]).wait()

        def grp(g, _):
            c16 = cl_b[b][pl.ds(g * L, L)] - nbase
            ok = (c16 >= 0) & (c16 < NH)
            cv_b[b][pl.ds(g * L, L)] = jnp.where(ok, c16, NH)
            r16 = rowv_all[pl.ds(c * K + g * L, L)]
            disr = plsc.load_gather(disv, [r16])
            w16 = el_b[b][pl.ds(g * L, L)] * disr
            for j in range(L):
                e = g * L + j
                w = w16[j]
                for f in range(H // L):
                    rows_b[b][e, pl.ds(f * L, L)] = (
                        rows_b[b][e, pl.ds(f * L, L)] * w)
            return 0
        lax.fori_loop(0, K // L, grp, 0)
        pltpu.async_copy(rows_b[b], accsh.at[cv_b[b]], ss_b[b], add=True)

        bm = (b + 2) % DEPTH

        @pl.when(c >= 1)
        def _():
            pltpu.make_async_copy(rows_b[bm], accsh.at[cv_b[bm]],
                                  ss_b[bm]).wait()

        @pl.when(c + 2 < NCH)
        def _():
            issue_g(c + 2, bm)

    issue_g(0, 0)
    issue_g(1, 1)

    def main(t, _):
        c0 = t * DEPTH
        for off in range(DEPTH):
            chunk(c0 + off, off)
        return 0
    lax.fori_loop(0, (NCH - 1) // DEPTH, main, 0)
    chunk(NCH - 1, (NCH - 1) % DEPTH)
    pltpu.make_async_copy(rows_b[(NCH - 1) % DEPTH],
                          accsh.at[cv_b[(NCH - 1) % DEPTH]],
                          ss_b[(NCH - 1) % DEPTH]).wait()

    plsc.subcore_barrier()

    cp = NH // 8

    @pl.when(sid < 8)
    def _():
        pltpu.sync_copy(accsh.at[pl.ds(sid * cp, cp)],
                        out_hbm.at[pl.ds(nbase + sid * cp, cp)])


def _msg_partials(y, disf, row, col, ew):
    mesh = plsc.VectorSubcoreMesh(core_axis_name="c", subcore_axis_name="s",
                                  num_cores=NC, num_subcores=NS)
    return pl.kernel(
        _msg_body,
        out_type=jax.ShapeDtypeStruct((NPAD, H), jnp.float32),
        mesh=mesh,
        scratch_types=[
            pltpu.VMEM((ET1,), jnp.int32),
            pltpu.VMEM((NPAD,), jnp.float32),
            pltpu.VMEM((K,), jnp.int32),
            pltpu.VMEM((K,), jnp.int32),
            pltpu.VMEM((K,), jnp.int32),
            pltpu.VMEM((K,), jnp.float32),
            pltpu.VMEM((K,), jnp.float32),
            pltpu.VMEM((K,), jnp.float32),
            pltpu.VMEM((K, H), jnp.float32),
            pltpu.VMEM((K, H), jnp.float32),
            pltpu.VMEM((K, H), jnp.float32),
            pltpu.VMEM((K,), jnp.int32),
            pltpu.VMEM((K,), jnp.int32),
            pltpu.VMEM((K,), jnp.int32),
            pltpu.VMEM((8, H), jnp.float32),
            pltpu.VMEM_SHARED((AR, H), jnp.float32),
            pltpu.SemaphoreType.DMA,
            pltpu.SemaphoreType.DMA,
            pltpu.SemaphoreType.DMA,
            pltpu.SemaphoreType.DMA,
            pltpu.SemaphoreType.DMA,
            pltpu.SemaphoreType.DMA,
            pltpu.SemaphoreType.DMA,
            pltpu.SemaphoreType.DMA,
            pltpu.SemaphoreType.DMA,
        ],
        compiler_params=_SC_PARAMS,
    )(y, disf, row, col, ew)


def _final_body(disbc_ref, s_ref, xw_ref, b1_ref, batch_ref,
                w2_ref, b2_ref, w3_ref, b3_ref, out_ref, pool, cnt):
    i = pl.program_id(0)

    @pl.when(i == 0)
    def _():
        pool[...] = jnp.zeros_like(pool)
        cnt[...] = jnp.zeros_like(cnt)

    dis = disbc_ref[...]
    x1 = jnp.maximum((s_ref[...] + xw_ref[...] * dis) * dis + b1_ref[...],
                     0.0)

    oh = (batch_ref[0, 0][:, None]
          == lax.broadcasted_iota(jnp.int32, (BN, G), 1)).astype(jnp.float32)
    pool[...] += jnp.dot(oh.T, x1, preferred_element_type=jnp.float32)
    cnt[...] += jnp.dot(oh.T, jnp.ones_like(x1),
                        preferred_element_type=jnp.float32)

    @pl.when(i == NB - 1)
    def _():
        pooled = pool[...] / jnp.maximum(cnt[...], 1.0)
        x2 = jnp.maximum(jnp.dot(pooled, w2_ref[...],
                                 preferred_element_type=jnp.float32)
                         + b2_ref[...], 0.0)
        out_ref[...] = jnp.dot(x2, w3_ref[...],
                               preferred_element_type=jnp.float32) + b3_ref[...]


def _final(disbc, s_part, xw, b1, batch3, W2, b2, W3p, b3p):
    return pl.pallas_call(
        _final_body,
        grid=(NB,),
        in_specs=[
            pl.BlockSpec((BN, H), lambda i: (i, 0)),
            pl.BlockSpec((BN, H), lambda i: (i, 0)),
            pl.BlockSpec((BN, H), lambda i: (i, 0)),
            pl.BlockSpec((1, H), lambda i: (0, 0)),
            pl.BlockSpec((1, 1, BN), lambda i: (i, 0, 0)),
            pl.BlockSpec((H, H), lambda i: (0, 0)),
            pl.BlockSpec((1, H), lambda i: (0, 0)),
            pl.BlockSpec((H, H), lambda i: (0, 0)),
            pl.BlockSpec((1, H), lambda i: (0, 0)),
        ],
        out_specs=pl.BlockSpec((G, H), lambda i: (0, 0)),
        out_shape=jax.ShapeDtypeStruct((G, H), jnp.float32),
        scratch_shapes=[
            pltpu.VMEM((G, H), jnp.float32),
            pltpu.VMEM((G, H), jnp.float32),
        ],
    )(disbc, s_part, xw, b1, batch3, W2, b2, W3p, b3p)


def kernel(x, edge_index, edge_attr, batch, W1, b1, W2, b2, W3, b3,
           We1, be1, We2, be2):
    row = edge_index[0]
    col = edge_index[1]
    rowf = row.astype(jnp.float32).reshape(E // 128, 128)
    colf = col.astype(jnp.float32).reshape(E // 128, 128)
    attrf = edge_attr.reshape(E // 128, 128)
    params = jnp.concatenate(
        [We1[:, 0], be1, We2[0], be2,
         jnp.zeros((122,), jnp.float32)]).reshape(1, 128)

    x_pad = jnp.pad(x, ((0, NPAD - N), (0, 0)))
    ew2d, xw = _edge_weights_xw(rowf, colf, attrf, params, x_pad, W1)
    ew = ew2d.reshape(E)

    disbc, disf = _dis_broadcast(ew, col)
    s_part = _msg_partials(xw, disf, row, col, ew)

    batch3 = jnp.pad(batch, (0, NPAD - N),
                     constant_values=G).reshape(NB, 1, BN)
    W3p = jnp.pad(W3, ((0, 0), (0, H - A)))
    b3p = jnp.pad(b3, (0, H - A)).reshape(1, H)
    out = _final(disbc, s_part, xw, b1.reshape(1, H), batch3,
                 W2, b2.reshape(1, H), W3p, b3p)
    return out[:, :A]


# speedup vs baseline: 20.0274x; 1.0344x over previous
import functools

import jax
import jax.numpy as jnp
from jax import lax
from jax.experimental import pallas as pl
from jax.experimental.pallas import tpu as pltpu
from jax.experimental.pallas import tpu_sc as plsc

N = 10000
E = 320000
D = 128
H = 128
A = 32
G = 16

NC = 2
NS = 16
L = 16

NPT = 640
NPAD = NS * NPT
ET1 = E // NS
KD = 2000
K = 80
NH = NPAD // NC
AR = 5248
ZR = AR // NS
DR = NH // NS

NB = 10
BN = NPAD // NB

_SC_PARAMS = pltpu.CompilerParams(needs_layout_passes=False)


def _ew_body(rowf_ref, colf_ref, attr_ref, p_ref, x_ref, w1_ref,
             out_ref, xw_ref):
    p = p_ref[0]
    h = jnp.maximum(rowf_ref[...] * p[0] + colf_ref[...] * p[1]
                    + attr_ref[...] * p[2] + p[3], 0.0)
    out_ref[...] = jax.nn.sigmoid(h * p[4] + p[5])
    xw_ref[...] = jnp.dot(x_ref[...], w1_ref[...],
                          preferred_element_type=jnp.float32)


def _edge_weights_xw(rowf, colf, attrf, params, x_pad, W1):
    return pl.pallas_call(
        _ew_body,
        out_shape=[jax.ShapeDtypeStruct((E // 128, 128), jnp.float32),
                   jax.ShapeDtypeStruct((NPAD, H), jnp.float32)],
    )(rowf, colf, attrf, params, x_pad, W1)


def _deg_body(ew_hbm, col_hbm, disbc_hbm, disf_hbm,
              colv, ewv, acc, tmpv, dredv, drv, dbuf, redsh, dissh, sem):
    cid = lax.axis_index("c")
    sid = lax.axis_index("s")
    zero16 = jnp.zeros((L,), jnp.float32)

    def zg(g, _):
        acc[pl.ds(g * L, L)] = zero16
        return 0
    lax.fori_loop(0, NPAD // L, zg, 0)

    pltpu.sync_copy(ew_hbm.at[pl.ds(sid * ET1, ET1)], ewv)
    pltpu.sync_copy(col_hbm.at[pl.ds(sid * ET1, ET1)], colv)

    def grp(g, _):
        c16 = colv[pl.ds(g * L, L)]
        w16 = ewv[pl.ds(g * L, L)]
        plsc.addupdate_scatter(acc, [c16], w16)
        return 0
    lax.fori_loop(0, ET1 // L, grp, 0)

    pltpu.sync_copy(acc, redsh.at[sid])
    plsc.subcore_barrier()

    def zg2(g, _):
        dredv[pl.ds(g * L, L)] = zero16
        return 0
    lax.fori_loop(0, NPT // L, zg2, 0)
    for t in range(NS):
        pltpu.sync_copy(redsh.at[t, pl.ds(sid * NPT, NPT)], tmpv)

        def addg(g, _):
            dredv[pl.ds(g * L, L)] = (dredv[pl.ds(g * L, L)]
                                      + tmpv[pl.ds(g * L, L)])
            return 0
        lax.fori_loop(0, NPT // L, addg, 0)

    def newt(g, _):
        d = 1.0 + dredv[pl.ds(g * L, L)]
        bits = plsc.bitcast(d, jnp.int32)
        yv = plsc.bitcast(jnp.int32(0x5F3759DF)
                          - lax.shift_right_logical(bits, 1), jnp.float32)
        for _ in range(3):
            yv = yv * (1.5 - 0.5 * d * yv * yv)
        dredv[pl.ds(g * L, L)] = yv
        return 0
    lax.fori_loop(0, NPT // L, newt, 0)

    pltpu.sync_copy(dredv, dissh.at[pl.ds(sid * NPT, NPT)])

    @pl.when(cid == 0)
    def _():
        pltpu.sync_copy(dredv, disf_hbm.at[pl.ds(sid * NPT, NPT)])
    plsc.subcore_barrier()

    obase = cid * NH + sid * DR
    pltpu.sync_copy(dissh.at[pl.ds(obase, DR)], drv)

    def brow(g, _):
        d16 = drv[pl.ds(g * L, L)]
        for j in range(L):
            r = g * L + j
            dj = jnp.full((L,), d16[j], jnp.float32)
            for f in range(D // L):
                dbuf[r, pl.ds(f * L, L)] = dj
        return 0
    lax.fori_loop(0, DR // L, brow, 0)
    pltpu.sync_copy(dbuf, disbc_hbm.at[pl.ds(obase, DR)])


def _dis_broadcast(ew, col):
    mesh = plsc.VectorSubcoreMesh(core_axis_name="c", subcore_axis_name="s",
                                  num_cores=NC, num_subcores=NS)
    return pl.kernel(
        _deg_body,
        out_type=[jax.ShapeDtypeStruct((NPAD, D), jnp.float32),
                  jax.ShapeDtypeStruct((NPAD,), jnp.float32)],
        mesh=mesh,
        scratch_types=[
            pltpu.VMEM((ET1,), jnp.int32),
            pltpu.VMEM((ET1,), jnp.float32),
            pltpu.VMEM((NPAD,), jnp.float32),
            pltpu.VMEM((NPT,), jnp.float32),
            pltpu.VMEM((NPT,), jnp.float32),
            pltpu.VMEM((DR,), jnp.float32),
            pltpu.VMEM((DR, D), jnp.float32),
            pltpu.VMEM_SHARED((NS, NPAD), jnp.float32),
            pltpu.VMEM_SHARED((NPAD,), jnp.float32),
            pltpu.SemaphoreType.DMA,
        ],
        compiler_params=_SC_PARAMS,
    )(ew, col)


NCH = ET1 // K
DEPTH = 3


def _msg_body(y_hbm, disf_hbm, row_hbm, col_hbm, ew_hbm, out_hbm,
              rowv_all, disv, cl0, cl1, cl2, el0, el1, el2,
              rows0, rows1, rows2, cv0, cv1, cv2, zbuf, accsh,
              gs0, gs1, gs2, ss0, ss1, ss2, es0, es1, es2):
    cid = lax.axis_index("c")
    sid = lax.axis_index("s")
    zero16 = jnp.zeros((L,), jnp.float32)
    nbase = cid * NH
    ebase = sid * ET1

    rows_b = (rows0, rows1, rows2)
    cv_b = (cv0, cv1, cv2)
    cl_b = (cl0, cl1, cl2)
    el_b = (el0, el1, el2)
    gs_b = (gs0, gs1, gs2)
    ss_b = (ss0, ss1, ss2)
    es_b = (es0, es1, es2)

    for r in range(40):
        for f in range(H // L):
            zbuf[r, pl.ds(f * L, L)] = zero16

    def zcp(i, _):
        pltpu.sync_copy(zbuf, accsh.at[pl.ds(sid * ZR + i * 40, 40)])
        return 0
    lax.fori_loop(0, ZR // 40, zcp, 0)
    pltpu.sync_copy(zbuf.at[pl.ds(0, 8)],
                    accsh.at[pl.ds(sid * ZR + ZR - 8, 8)])

    pltpu.sync_copy(row_hbm.at[pl.ds(ebase, ET1)], rowv_all)
    pltpu.sync_copy(disf_hbm, disv)
    plsc.subcore_barrier()

    def issue_g(c, b):
        pltpu.async_copy(y_hbm.at[rowv_all.at[pl.ds(c * K, K)]],
                         rows_b[b], gs_b[b])
        pltpu.async_copy(col_hbm.at[pl.ds(ebase + c * K, K)], cl_b[b], es_b[b])
        pltpu.async_copy(ew_hbm.at[pl.ds(ebase + c * K, K)], el_b[b], es_b[b])

    def chunk(c, b):
        pltpu.make_async_copy(y_hbm.at[rowv_all.at[pl.ds(c * K, K)]],
                              rows_b[b], gs_b[b]).wait()
        cb = ebase + c * K
        pltpu.make_async_copy(col_hbm.at[pl.ds(cb, K)], cl_b[b],
                              es_b[b]).wait()
        pltpu.make_async_copy(ew_hbm.at[pl.ds(cb, K)], el_b[b],
                              es_b[b]).wait()

        def grp(g, _):
            c16 = cl_b[b][pl.ds(g * L, L)] - nbase
            ok = (c16 >= 0) & (c16 < NH)
            cv_b[b][pl.ds(g * L, L)] = jnp.where(ok, c16, NH)
            r16 = rowv_all[pl.ds(c * K + g * L, L)]
            disr = plsc.load_gather(disv, [r16])
            w16 = el_b[b][pl.ds(g * L, L)] * disr
            ok32 = jnp.where(ok, 1, 0)
            for j in range(L):
                e = g * L + j

                @pl.when(ok32[j] > 0)
                def _():
                    w = w16[j]
                    for f in range(H // L):
                        rows_b[b][e, pl.ds(f * L, L)] = (
                            rows_b[b][e, pl.ds(f * L, L)] * w)
            return 0
        lax.fori_loop(0, K // L, grp, 0)
        pltpu.async_copy(rows_b[b], accsh.at[cv_b[b]], ss_b[b], add=True)

        bm = (b + 2) % DEPTH

        @pl.when(c >= 1)
        def _():
            pltpu.make_async_copy(rows_b[bm], accsh.at[cv_b[bm]],
                                  ss_b[bm]).wait()

        @pl.when(c + 2 < NCH)
        def _():
            issue_g(c + 2, bm)

    issue_g(0, 0)
    issue_g(1, 1)

    def main(t, _):
        c0 = t * DEPTH
        for off in range(DEPTH):
            chunk(c0 + off, off)
        return 0
    lax.fori_loop(0, (NCH - 1) // DEPTH, main, 0)
    chunk(NCH - 1, (NCH - 1) % DEPTH)
    pltpu.make_async_copy(rows_b[(NCH - 1) % DEPTH],
                          accsh.at[cv_b[(NCH - 1) % DEPTH]],
                          ss_b[(NCH - 1) % DEPTH]).wait()

    plsc.subcore_barrier()

    cp = NH // 8

    @pl.when(sid < 8)
    def _():
        pltpu.sync_copy(accsh.at[pl.ds(sid * cp, cp)],
                        out_hbm.at[pl.ds(nbase + sid * cp, cp)])


def _msg_partials(y, disf, row, col, ew):
    mesh = plsc.VectorSubcoreMesh(core_axis_name="c", subcore_axis_name="s",
                                  num_cores=NC, num_subcores=NS)
    return pl.kernel(
        _msg_body,
        out_type=jax.ShapeDtypeStruct((NPAD, H), jnp.float32),
        mesh=mesh,
        scratch_types=[
            pltpu.VMEM((ET1,), jnp.int32),
            pltpu.VMEM((NPAD,), jnp.float32),
            pltpu.VMEM((K,), jnp.int32),
            pltpu.VMEM((K,), jnp.int32),
            pltpu.VMEM((K,), jnp.int32),
            pltpu.VMEM((K,), jnp.float32),
            pltpu.VMEM((K,), jnp.float32),
            pltpu.VMEM((K,), jnp.float32),
            pltpu.VMEM((K, H), jnp.float32),
            pltpu.VMEM((K, H), jnp.float32),
            pltpu.VMEM((K, H), jnp.float32),
            pltpu.VMEM((K,), jnp.int32),
            pltpu.VMEM((K,), jnp.int32),
            pltpu.VMEM((K,), jnp.int32),
            pltpu.VMEM((40, H), jnp.float32),
            pltpu.VMEM_SHARED((AR, H), jnp.float32),
            pltpu.SemaphoreType.DMA,
            pltpu.SemaphoreType.DMA,
            pltpu.SemaphoreType.DMA,
            pltpu.SemaphoreType.DMA,
            pltpu.SemaphoreType.DMA,
            pltpu.SemaphoreType.DMA,
            pltpu.SemaphoreType.DMA,
            pltpu.SemaphoreType.DMA,
            pltpu.SemaphoreType.DMA,
        ],
        compiler_params=_SC_PARAMS,
    )(y, disf, row, col, ew)


def _final_body(disbc_ref, s_ref, xw_ref, b1_ref, batch_ref,
                w2_ref, b2_ref, w3_ref, b3_ref, out_ref, pool, cnt):
    i = pl.program_id(0)

    @pl.when(i == 0)
    def _():
        pool[...] = jnp.zeros_like(pool)
        cnt[...] = jnp.zeros_like(cnt)

    dis = disbc_ref[...]
    x1 = jnp.maximum((s_ref[...] + xw_ref[...] * dis) * dis + b1_ref[...],
                     0.0)

    oh = (batch_ref[0, 0][:, None]
          == lax.broadcasted_iota(jnp.int32, (BN, G), 1)).astype(jnp.float32)
    pool[...] += jnp.dot(oh.T, x1, preferred_element_type=jnp.float32)
    cnt[...] += jnp.dot(oh.T, jnp.ones_like(x1),
                        preferred_element_type=jnp.float32)

    @pl.when(i == NB - 1)
    def _():
        pooled = pool[...] / jnp.maximum(cnt[...], 1.0)
        x2 = jnp.maximum(jnp.dot(pooled, w2_ref[...],
                                 preferred_element_type=jnp.float32)
                         + b2_ref[...], 0.0)
        out_ref[...] = jnp.dot(x2, w3_ref[...],
                               preferred_element_type=jnp.float32) + b3_ref[...]


def _final(disbc, s_part, xw, b1, batch3, W2, b2, W3p, b3p):
    return pl.pallas_call(
        _final_body,
        grid=(NB,),
        in_specs=[
            pl.BlockSpec((BN, H), lambda i: (i, 0)),
            pl.BlockSpec((BN, H), lambda i: (i, 0)),
            pl.BlockSpec((BN, H), lambda i: (i, 0)),
            pl.BlockSpec((1, H), lambda i: (0, 0)),
            pl.BlockSpec((1, 1, BN), lambda i: (i, 0, 0)),
            pl.BlockSpec((H, H), lambda i: (0, 0)),
            pl.BlockSpec((1, H), lambda i: (0, 0)),
            pl.BlockSpec((H, H), lambda i: (0, 0)),
            pl.BlockSpec((1, H), lambda i: (0, 0)),
        ],
        out_specs=pl.BlockSpec((G, H), lambda i: (0, 0)),
        out_shape=jax.ShapeDtypeStruct((G, H), jnp.float32),
        scratch_shapes=[
            pltpu.VMEM((G, H), jnp.float32),
            pltpu.VMEM((G, H), jnp.float32),
        ],
    )(disbc, s_part, xw, b1, batch3, W2, b2, W3p, b3p)


def kernel(x, edge_index, edge_attr, batch, W1, b1, W2, b2, W3, b3,
           We1, be1, We2, be2):
    row = edge_index[0]
    col = edge_index[1]
    rowf = row.astype(jnp.float32).reshape(E // 128, 128)
    colf = col.astype(jnp.float32).reshape(E // 128, 128)
    attrf = edge_attr.reshape(E // 128, 128)
    params = jnp.concatenate(
        [We1[:, 0], be1, We2[0], be2,
         jnp.zeros((122,), jnp.float32)]).reshape(1, 128)

    x_pad = jnp.pad(x, ((0, NPAD - N), (0, 0)))
    ew2d, xw = _edge_weights_xw(rowf, colf, attrf, params, x_pad, W1)
    ew = ew2d.reshape(E)

    disbc, disf = _dis_broadcast(ew, col)
    s_part = _msg_partials(xw, disf, row, col, ew)

    batch3 = jnp.pad(batch, (0, NPAD - N),
                     constant_values=G).reshape(NB, 1, BN)
    W3p = jnp.pad(W3, ((0, 0), (0, H - A)))
    b3p = jnp.pad(b3, (0, H - A)).reshape(1, H)
    out = _final(disbc, s_part, xw, b1.reshape(1, H), batch3,
                 W2, b2.reshape(1, H), W3p, b3p)
    return out[:, :A]


# speedup vs baseline: 20.1876x; 1.0080x over previous
import functools

import jax
import jax.numpy as jnp
from jax import lax
from jax.experimental import pallas as pl
from jax.experimental.pallas import tpu as pltpu
from jax.experimental.pallas import tpu_sc as plsc

N = 10000
E = 320000
D = 128
H = 128
A = 32
G = 16

NC = 2
NS = 16
L = 16

NPT = 640
NPAD = NS * NPT
ET1 = E // NS
KD = 2000
K = 80
NH = NPAD // NC
AR = 5248
ZR = AR // NS
DR = NH // NS

NB = 10
BN = NPAD // NB

_SC_PARAMS = pltpu.CompilerParams(needs_layout_passes=False)


def _ew_body(rowf_ref, colf_ref, attr_ref, p_ref, x_ref, w1_ref,
             out_ref, xw_ref):
    p = p_ref[0]
    h = jnp.maximum(rowf_ref[...] * p[0] + colf_ref[...] * p[1]
                    + attr_ref[...] * p[2] + p[3], 0.0)
    out_ref[...] = jax.nn.sigmoid(h * p[4] + p[5])
    xw_ref[:N] = jnp.dot(x_ref[...], w1_ref[...],
                         preferred_element_type=jnp.float32)
    xw_ref[N:] = jnp.zeros((NPAD - N, H), jnp.float32)


def _edge_weights_xw(rowf, colf, attrf, params, x_pad, W1):
    return pl.pallas_call(
        _ew_body,
        out_shape=[jax.ShapeDtypeStruct((E // 128, 128), jnp.float32),
                   jax.ShapeDtypeStruct((NPAD, H), jnp.float32)],
    )(rowf, colf, attrf, params, x_pad, W1)


def _deg_body(ew_hbm, col_hbm, disbc_hbm, disf_hbm,
              colv, ewv, acc, tmpv, dredv, drv, dbuf, redsh, dissh, sem):
    cid = lax.axis_index("c")
    sid = lax.axis_index("s")
    zero16 = jnp.zeros((L,), jnp.float32)

    def zg(g, _):
        acc[pl.ds(g * L, L)] = zero16
        return 0
    lax.fori_loop(0, NPAD // L, zg, 0)

    pltpu.sync_copy(ew_hbm.at[pl.ds(sid * ET1, ET1)], ewv)
    pltpu.sync_copy(col_hbm.at[pl.ds(sid * ET1, ET1)], colv)

    def grp(g, _):
        c16 = colv[pl.ds(g * L, L)]
        w16 = ewv[pl.ds(g * L, L)]
        plsc.addupdate_scatter(acc, [c16], w16)
        return 0
    lax.fori_loop(0, ET1 // L, grp, 0)

    pltpu.sync_copy(acc, redsh.at[sid])
    plsc.subcore_barrier()

    def zg2(g, _):
        dredv[pl.ds(g * L, L)] = zero16
        return 0
    lax.fori_loop(0, NPT // L, zg2, 0)
    for t in range(NS):
        pltpu.sync_copy(redsh.at[t, pl.ds(sid * NPT, NPT)], tmpv)

        def addg(g, _):
            dredv[pl.ds(g * L, L)] = (dredv[pl.ds(g * L, L)]
                                      + tmpv[pl.ds(g * L, L)])
            return 0
        lax.fori_loop(0, NPT // L, addg, 0)

    def newt(g, _):
        d = 1.0 + dredv[pl.ds(g * L, L)]
        bits = plsc.bitcast(d, jnp.int32)
        yv = plsc.bitcast(jnp.int32(0x5F3759DF)
                          - lax.shift_right_logical(bits, 1), jnp.float32)
        for _ in range(3):
            yv = yv * (1.5 - 0.5 * d * yv * yv)
        dredv[pl.ds(g * L, L)] = yv
        return 0
    lax.fori_loop(0, NPT // L, newt, 0)

    pltpu.sync_copy(dredv, dissh.at[pl.ds(sid * NPT, NPT)])

    @pl.when(cid == 0)
    def _():
        pltpu.sync_copy(dredv, disf_hbm.at[pl.ds(sid * NPT, NPT)])
    plsc.subcore_barrier()

    obase = cid * NH + sid * DR
    pltpu.sync_copy(dissh.at[pl.ds(obase, DR)], drv)

    def brow(g, _):
        d16 = drv[pl.ds(g * L, L)]
        for j in range(L):
            r = g * L + j
            dj = jnp.full((L,), d16[j], jnp.float32)
            for f in range(D // L):
                dbuf[r, pl.ds(f * L, L)] = dj
        return 0
    lax.fori_loop(0, DR // L, brow, 0)
    pltpu.sync_copy(dbuf, disbc_hbm.at[pl.ds(obase, DR)])


def _dis_broadcast(ew, col):
    mesh = plsc.VectorSubcoreMesh(core_axis_name="c", subcore_axis_name="s",
                                  num_cores=NC, num_subcores=NS)
    return pl.kernel(
        _deg_body,
        out_type=[jax.ShapeDtypeStruct((NPAD, D), jnp.float32),
                  jax.ShapeDtypeStruct((NPAD,), jnp.float32)],
        mesh=mesh,
        scratch_types=[
            pltpu.VMEM((ET1,), jnp.int32),
            pltpu.VMEM((ET1,), jnp.float32),
            pltpu.VMEM((NPAD,), jnp.float32),
            pltpu.VMEM((NPT,), jnp.float32),
            pltpu.VMEM((NPT,), jnp.float32),
            pltpu.VMEM((DR,), jnp.float32),
            pltpu.VMEM((DR, D), jnp.float32),
            pltpu.VMEM_SHARED((NS, NPAD), jnp.float32),
            pltpu.VMEM_SHARED((NPAD,), jnp.float32),
            pltpu.SemaphoreType.DMA,
        ],
        compiler_params=_SC_PARAMS,
    )(ew, col)


NCH = ET1 // K
DEPTH = 3


def _msg_body(y_hbm, disf_hbm, row_hbm, col_hbm, ew_hbm, out_hbm,
              rowv_all, disv, cl0, cl1, cl2, el0, el1, el2,
              rows0, rows1, rows2, cv0, cv1, cv2, zbuf, accsh,
              gs0, gs1, gs2, ss0, ss1, ss2, es0, es1, es2):
    cid = lax.axis_index("c")
    sid = lax.axis_index("s")
    zero16 = jnp.zeros((L,), jnp.float32)
    nbase = cid * NH
    ebase = sid * ET1

    rows_b = (rows0, rows1, rows2)
    cv_b = (cv0, cv1, cv2)
    cl_b = (cl0, cl1, cl2)
    el_b = (el0, el1, el2)
    gs_b = (gs0, gs1, gs2)
    ss_b = (ss0, ss1, ss2)
    es_b = (es0, es1, es2)

    for r in range(40):
        for f in range(H // L):
            zbuf[r, pl.ds(f * L, L)] = zero16

    def zcp(i, _):
        pltpu.sync_copy(zbuf, accsh.at[pl.ds(sid * ZR + i * 40, 40)])
        return 0
    lax.fori_loop(0, ZR // 40, zcp, 0)
    pltpu.sync_copy(zbuf.at[pl.ds(0, 8)],
                    accsh.at[pl.ds(sid * ZR + ZR - 8, 8)])

    pltpu.sync_copy(row_hbm.at[pl.ds(ebase, ET1)], rowv_all)
    pltpu.sync_copy(disf_hbm, disv)
    plsc.subcore_barrier()

    def issue_g(c, b):
        pltpu.async_copy(y_hbm.at[rowv_all.at[pl.ds(c * K, K)]],
                         rows_b[b], gs_b[b])
        pltpu.async_copy(col_hbm.at[pl.ds(ebase + c * K, K)], cl_b[b], es_b[b])
        pltpu.async_copy(ew_hbm.at[pl.ds(ebase + c * K, K)], el_b[b], es_b[b])

    def chunk(c, b):
        pltpu.make_async_copy(y_hbm.at[rowv_all.at[pl.ds(c * K, K)]],
                              rows_b[b], gs_b[b]).wait()
        cb = ebase + c * K
        pltpu.make_async_copy(col_hbm.at[pl.ds(cb, K)], cl_b[b],
                              es_b[b]).wait()
        pltpu.make_async_copy(ew_hbm.at[pl.ds(cb, K)], el_b[b],
                              es_b[b]).wait()

        def grp(g, _):
            c16 = cl_b[b][pl.ds(g * L, L)] - nbase
            ok = (c16 >= 0) & (c16 < NH)
            cv_b[b][pl.ds(g * L, L)] = jnp.where(ok, c16, NH)
            r16 = rowv_all[pl.ds(c * K + g * L, L)]
            disr = plsc.load_gather(disv, [r16])
            w16 = el_b[b][pl.ds(g * L, L)] * disr
            ok32 = jnp.where(ok, 1, 0)
            for j in range(L):
                e = g * L + j

                @pl.when(ok32[j] > 0)
                def _():
                    w = w16[j]
                    for f in range(H // L):
                        rows_b[b][e, pl.ds(f * L, L)] = (
                            rows_b[b][e, pl.ds(f * L, L)] * w)
            return 0
        lax.fori_loop(0, K // L, grp, 0)
        pltpu.async_copy(rows_b[b], accsh.at[cv_b[b]], ss_b[b], add=True)

        bm = (b + 2) % DEPTH

        @pl.when(c >= 1)
        def _():
            pltpu.make_async_copy(rows_b[bm], accsh.at[cv_b[bm]],
                                  ss_b[bm]).wait()

        @pl.when(c + 2 < NCH)
        def _():
            issue_g(c + 2, bm)

    issue_g(0, 0)
    issue_g(1, 1)

    def main(t, _):
        c0 = t * DEPTH
        for off in range(DEPTH):
            chunk(c0 + off, off)
        return 0
    lax.fori_loop(0, (NCH - 1) // DEPTH, main, 0)
    chunk(NCH - 1, (NCH - 1) % DEPTH)
    pltpu.make_async_copy(rows_b[(NCH - 1) % DEPTH],
                          accsh.at[cv_b[(NCH - 1) % DEPTH]],
                          ss_b[(NCH - 1) % DEPTH]).wait()

    plsc.subcore_barrier()

    cp = NH // 8

    @pl.when(sid < 8)
    def _():
        pltpu.sync_copy(accsh.at[pl.ds(sid * cp, cp)],
                        out_hbm.at[pl.ds(nbase + sid * cp, cp)])


def _msg_partials(y, disf, row, col, ew):
    mesh = plsc.VectorSubcoreMesh(core_axis_name="c", subcore_axis_name="s",
                                  num_cores=NC, num_subcores=NS)
    return pl.kernel(
        _msg_body,
        out_type=jax.ShapeDtypeStruct((NPAD, H), jnp.float32),
        mesh=mesh,
        scratch_types=[
            pltpu.VMEM((ET1,), jnp.int32),
            pltpu.VMEM((NPAD,), jnp.float32),
            pltpu.VMEM((K,), jnp.int32),
            pltpu.VMEM((K,), jnp.int32),
            pltpu.VMEM((K,), jnp.int32),
            pltpu.VMEM((K,), jnp.float32),
            pltpu.VMEM((K,), jnp.float32),
            pltpu.VMEM((K,), jnp.float32),
            pltpu.VMEM((K, H), jnp.float32),
            pltpu.VMEM((K, H), jnp.float32),
            pltpu.VMEM((K, H), jnp.float32),
            pltpu.VMEM((K,), jnp.int32),
            pltpu.VMEM((K,), jnp.int32),
            pltpu.VMEM((K,), jnp.int32),
            pltpu.VMEM((40, H), jnp.float32),
            pltpu.VMEM_SHARED((AR, H), jnp.float32),
            pltpu.SemaphoreType.DMA,
            pltpu.SemaphoreType.DMA,
            pltpu.SemaphoreType.DMA,
            pltpu.SemaphoreType.DMA,
            pltpu.SemaphoreType.DMA,
            pltpu.SemaphoreType.DMA,
            pltpu.SemaphoreType.DMA,
            pltpu.SemaphoreType.DMA,
            pltpu.SemaphoreType.DMA,
        ],
        compiler_params=_SC_PARAMS,
    )(y, disf, row, col, ew)


def _final_body(disbc_ref, s_ref, xw_ref, b1_ref, batch_ref,
                w2_ref, b2_ref, w3_ref, b3_ref, out_ref, pool, cnt):
    i = pl.program_id(0)

    @pl.when(i == 0)
    def _():
        pool[...] = jnp.zeros_like(pool)
        cnt[...] = jnp.zeros_like(cnt)

    dis = disbc_ref[...]
    x1 = jnp.maximum((s_ref[...] + xw_ref[...] * dis) * dis + b1_ref[...],
                     0.0)

    oh = (batch_ref[0, 0][:, None]
          == lax.broadcasted_iota(jnp.int32, (BN, G), 1)).astype(jnp.float32)
    dn = (((0,), (0,)), ((), ()))
    pool[...] += lax.dot_general(oh, x1, dn,
                                 preferred_element_type=jnp.float32)
    cnt[...] += lax.dot_general(oh, jnp.ones_like(x1), dn,
                                preferred_element_type=jnp.float32)

    @pl.when(i == NB - 1)
    def _():
        pooled = pool[...] / jnp.maximum(cnt[...], 1.0)
        x2 = jnp.maximum(jnp.dot(pooled, w2_ref[...],
                                 preferred_element_type=jnp.float32)
                         + b2_ref[...], 0.0)
        out_ref[...] = jnp.dot(x2, w3_ref[...],
                               preferred_element_type=jnp.float32) + b3_ref[...]


def _final(disbc, s_part, xw, b1, batch3, W2, b2, W3p, b3p):
    return pl.pallas_call(
        _final_body,
        grid=(NB,),
        in_specs=[
            pl.BlockSpec((BN, H), lambda i: (i, 0)),
            pl.BlockSpec((BN, H), lambda i: (i, 0)),
            pl.BlockSpec((BN, H), lambda i: (i, 0)),
            pl.BlockSpec((1, H), lambda i: (0, 0)),
            pl.BlockSpec((1, 1, BN), lambda i: (i, 0, 0)),
            pl.BlockSpec((H, H), lambda i: (0, 0)),
            pl.BlockSpec((1, H), lambda i: (0, 0)),
            pl.BlockSpec((H, H), lambda i: (0, 0)),
            pl.BlockSpec((1, H), lambda i: (0, 0)),
        ],
        out_specs=pl.BlockSpec((G, H), lambda i: (0, 0)),
        out_shape=jax.ShapeDtypeStruct((G, H), jnp.float32),
        scratch_shapes=[
            pltpu.VMEM((G, H), jnp.float32),
            pltpu.VMEM((G, H), jnp.float32),
        ],
    )(disbc, s_part, xw, b1, batch3, W2, b2, W3p, b3p)


def kernel(x, edge_index, edge_attr, batch, W1, b1, W2, b2, W3, b3,
           We1, be1, We2, be2):
    row = edge_index[0]
    col = edge_index[1]
    rowf = row.astype(jnp.float32).reshape(E // 128, 128)
    colf = col.astype(jnp.float32).reshape(E // 128, 128)
    attrf = edge_attr.reshape(E // 128, 128)
    params = jnp.concatenate(
        [We1[:, 0], be1, We2[0], be2,
         jnp.zeros((122,), jnp.float32)]).reshape(1, 128)

    ew2d, xw = _edge_weights_xw(rowf, colf, attrf, params, x, W1)
    ew = ew2d.reshape(E)

    disbc, disf = _dis_broadcast(ew, col)
    s_part = _msg_partials(xw, disf, row, col, ew)

    batch3 = jnp.pad(batch, (0, NPAD - N),
                     constant_values=G).reshape(NB, 1, BN)
    W3p = jnp.pad(W3, ((0, 0), (0, H - A)))
    b3p = jnp.pad(b3, (0, H - A)).reshape(1, H)
    out = _final(disbc, s_part, xw, b1.reshape(1, H), batch3,
                 W2, b2.reshape(1, H), W3p, b3p)
    return out[:, :A]
